# baseline (speedup 1.0000x reference)
import functools

import jax
import jax.numpy as jnp
from jax import lax
from jax.experimental import pallas as pl
from jax.experimental.pallas import tpu as pltpu
from jax.experimental.pallas import tpu_sc as plsc

_LANES = 16
_TILE = 128
_NC = 2
_NS = 16
_NW = _NC * _NS
_BLK = 6400


def _body(seq_hbm, out_hbm, inb, ovb, is0, is1, os0, os1,
          *, n, blk, nblocks, kmax):
    m = (n // _TILE) * _TILE
    nfull = m // blk
    tail = m - nfull * blk

    wid = lax.axis_index("s") * _NC + lax.axis_index("c")

    def bid(k):
        return wid + k * _NW

    def pred(k):
        return bid(k) < nblocks

    def is_tail(k):
        return bid(k) == nfull

    def start_in(k, p, isem):
        if tail:
            @pl.when(pred(k) & jnp.logical_not(is_tail(k)))
            def _():
                pltpu.async_copy(seq_hbm.at[pl.ds(bid(k) * blk, blk)],
                                 inb.at[p], isem)

            @pl.when(is_tail(k))
            def _():
                pltpu.async_copy(seq_hbm.at[pl.ds(nfull * blk, tail)],
                                 inb.at[p, pl.ds(0, tail)], isem)
        else:
            @pl.when(pred(k))
            def _():
                pltpu.async_copy(seq_hbm.at[pl.ds(bid(k) * blk, blk)],
                                 inb.at[p], isem)

    def wait_in(k, p, isem):
        if tail:
            @pl.when(jnp.logical_not(is_tail(k)))
            def _():
                pltpu.make_async_copy(seq_hbm.at[pl.ds(bid(k) * blk, blk)],
                                      inb.at[p], isem).wait()

            @pl.when(is_tail(k))
            def _():
                pltpu.make_async_copy(seq_hbm.at[pl.ds(nfull * blk, tail)],
                                      inb.at[p, pl.ds(0, tail)], isem).wait()
        else:
            pltpu.make_async_copy(seq_hbm.at[pl.ds(bid(k) * blk, blk)],
                                  inb.at[p], isem).wait()

    def start_out(k, p, osem):
        if tail:
            @pl.when(jnp.logical_not(is_tail(k)))
            def _():
                pltpu.async_copy(ovb.at[p],
                                 out_hbm.at[:, pl.ds(bid(k) * blk, blk)],
                                 osem)

            @pl.when(is_tail(k))
            def _():
                pltpu.async_copy(ovb.at[p, :, pl.ds(0, tail)],
                                 out_hbm.at[:, pl.ds(nfull * blk, tail)],
                                 osem)
        else:
            pltpu.async_copy(ovb.at[p],
                             out_hbm.at[:, pl.ds(bid(k) * blk, blk)], osem)

    def wait_out(k, p, osem):
        if tail:
            @pl.when(jnp.logical_not(is_tail(k)))
            def _():
                pltpu.make_async_copy(
                    ovb.at[p], out_hbm.at[:, pl.ds(bid(k) * blk, blk)],
                    osem).wait()

            @pl.when(is_tail(k))
            def _():
                pltpu.make_async_copy(
                    ovb.at[p, :, pl.ds(0, tail)],
                    out_hbm.at[:, pl.ds(nfull * blk, tail)], osem).wait()
        else:
            pltpu.make_async_copy(
                ovb.at[p], out_hbm.at[:, pl.ds(bid(k) * blk, blk)],
                osem).wait()

    def compute(p):
        one = jnp.full((_LANES,), 1.0, jnp.float32)
        quarter = jnp.full((_LANES,), 0.25, jnp.float32)
        zero = jnp.zeros((_LANES,), jnp.float32)

        @plsc.parallel_loop(0, blk, step=_LANES, unroll=8)
        def inner(i):
            off = pl.multiple_of(i, _LANES)
            s = inb[p, pl.ds(off, _LANES)]
            nv = jnp.where(s == 4, quarter, zero)
            for c in range(4):
                ovb[p, c, pl.ds(off, _LANES)] = jnp.where(s == c, one, nv)

    sems = ((is0, os0), (is1, os1))

    start_in(0, 0, is0)

    def step(k, carry):
        p = lax.rem(k, 2)

        for pp in range(2):
            @pl.when(p == pp)
            def _(pp=pp):
                start_in(k + 1, 1 - pp, sems[1 - pp][0])

                @pl.when(pred(k))
                def _():
                    wait_in(k, pp, sems[pp][0])

                    @pl.when(k >= 2)
                    def _():
                        wait_out(k - 2, pp, sems[pp][1])

        @pl.when(pred(k))
        def _():
            compute(p)

        for pp in range(2):
            @pl.when((p == pp) & pred(k))
            def _(pp=pp):
                start_out(k, pp, sems[pp][1])

        return carry

    lax.fori_loop(0, kmax, step, 0)

    for j in range(max(0, kmax - 3), kmax):
        @pl.when(pred(j) & jnp.logical_not(pred(j + 2)))
        def _(j=j):
            wait_out(j, j % 2, sems[j % 2][1])


def kernel(seq):
    n = seq.shape[0]
    blk = _BLK
    m = (n // _TILE) * _TILE
    nblocks = -(-m // blk)
    kmax = -(-nblocks // _NW)
    mesh = plsc.VectorSubcoreMesh(core_axis_name="c", subcore_axis_name="s")
    f = pl.kernel(
        functools.partial(_body, n=n, blk=blk, nblocks=nblocks, kmax=kmax),
        out_type=jax.ShapeDtypeStruct((4, n), jnp.float32),
        mesh=mesh,
        scratch_types=[pltpu.VMEM((2, blk), jnp.int32),
                       pltpu.VMEM((2, 4, blk), jnp.float32)]
        + [pltpu.SemaphoreType.DMA for _ in range(4)],
    )
    seq = seq.astype(jnp.int32)
    out = f(seq)
    if m < n:
        rem = seq[m:]
        cls = jnp.arange(4, dtype=jnp.int32)[:, None]
        patch = jnp.where(rem[None, :] == cls, jnp.float32(1.0),
                          jnp.where(rem[None, :] == 4,
                                    jnp.float32(0.25), jnp.float32(0.0)))
        out = lax.dynamic_update_slice(out, patch, (0, m))
    return out


# speedup vs baseline: 1.1062x; 1.1062x over previous
import functools

import jax
import jax.numpy as jnp
from jax import lax
from jax.experimental import pallas as pl
from jax.experimental.pallas import tpu as pltpu
from jax.experimental.pallas import tpu_sc as plsc

_LANES = 16
_TILE = 128
_NC = 2
_NS = 16
_NW = _NC * _NS
_BLK = 6400


def _body(seq_hbm, out_hbm, in0, in1, ov0, ov1, is0, is1, os0, os1,
          *, n, blk, nblocks, kmax):
    m = (n // _TILE) * _TILE
    nfull = m // blk
    tail = m - nfull * blk

    wid = lax.axis_index("s") * _NC + lax.axis_index("c")

    def bid(k):
        return wid + k * _NW

    def pred(k):
        return bid(k) < nblocks

    def is_tail(k):
        return bid(k) == nfull

    def start_in(k, iv, isem):
        if tail:
            @pl.when(pred(k) & jnp.logical_not(is_tail(k)))
            def _():
                pltpu.async_copy(seq_hbm.at[pl.ds(bid(k) * blk, blk)],
                                 iv, isem)

            @pl.when(is_tail(k))
            def _():
                pltpu.async_copy(seq_hbm.at[pl.ds(nfull * blk, tail)],
                                 iv.at[pl.ds(0, tail)], isem)
        else:
            @pl.when(pred(k))
            def _():
                pltpu.async_copy(seq_hbm.at[pl.ds(bid(k) * blk, blk)],
                                 iv, isem)

    def wait_in(k, iv, isem):
        if tail:
            @pl.when(jnp.logical_not(is_tail(k)))
            def _():
                pltpu.make_async_copy(seq_hbm.at[pl.ds(bid(k) * blk, blk)],
                                      iv, isem).wait()

            @pl.when(is_tail(k))
            def _():
                pltpu.make_async_copy(seq_hbm.at[pl.ds(nfull * blk, tail)],
                                      iv.at[pl.ds(0, tail)], isem).wait()
        else:
            pltpu.make_async_copy(seq_hbm.at[pl.ds(bid(k) * blk, blk)],
                                  iv, isem).wait()

    def start_out(k, ov, osem):
        if tail:
            @pl.when(jnp.logical_not(is_tail(k)))
            def _():
                pltpu.async_copy(ov, out_hbm.at[:, pl.ds(bid(k) * blk, blk)],
                                 osem)

            @pl.when(is_tail(k))
            def _():
                pltpu.async_copy(ov.at[:, pl.ds(0, tail)],
                                 out_hbm.at[:, pl.ds(nfull * blk, tail)],
                                 osem)
        else:
            pltpu.async_copy(ov, out_hbm.at[:, pl.ds(bid(k) * blk, blk)],
                             osem)

    def wait_out(k, ov, osem):
        if tail:
            @pl.when(jnp.logical_not(is_tail(k)))
            def _():
                pltpu.make_async_copy(
                    ov, out_hbm.at[:, pl.ds(bid(k) * blk, blk)], osem).wait()

            @pl.when(is_tail(k))
            def _():
                pltpu.make_async_copy(
                    ov.at[:, pl.ds(0, tail)],
                    out_hbm.at[:, pl.ds(nfull * blk, tail)], osem).wait()
        else:
            pltpu.make_async_copy(
                ov, out_hbm.at[:, pl.ds(bid(k) * blk, blk)], osem).wait()

    def compute(iv, ov):
        one = jnp.full((_LANES,), 1.0, jnp.float32)
        quarter = jnp.full((_LANES,), 0.25, jnp.float32)
        zero = jnp.zeros((_LANES,), jnp.float32)

        @plsc.parallel_loop(0, blk, step=_LANES, unroll=4)
        def inner(i):
            off = pl.multiple_of(i, _LANES)
            s = iv[pl.ds(off, _LANES)]
            nv = jnp.where(s == 4, quarter, zero)
            for c in range(4):
                ov[c, pl.ds(off, _LANES)] = jnp.where(s == c, one, nv)

    def iteration(k, cur, nxt):
        (iv, ov, isem, osem) = cur
        start_in(k + 1, nxt[0], nxt[2])

        @pl.when(pred(k))
        def _():
            wait_in(k, iv, isem)

            @pl.when(k >= 2)
            def _():
                wait_out(k - 2, ov, osem)

            compute(iv, ov)
            start_out(k, ov, osem)

    bufs = ((in0, ov0, is0, os0), (in1, ov1, is1, os1))

    start_in(0, in0, is0)

    def step(k, carry):
        @pl.when(lax.rem(k, 2) == 0)
        def _():
            iteration(k, bufs[0], bufs[1])

        @pl.when(lax.rem(k, 2) == 1)
        def _():
            iteration(k, bufs[1], bufs[0])

        return carry

    lax.fori_loop(0, kmax, step, 0)

    for j in range(max(0, kmax - 3), kmax):
        @pl.when(pred(j) & jnp.logical_not(pred(j + 2)))
        def _(j=j):
            wait_out(j, bufs[j % 2][1], bufs[j % 2][3])


def kernel(seq):
    n = seq.shape[0]
    blk = _BLK
    m = (n // _TILE) * _TILE
    nblocks = -(-m // blk)
    kmax = -(-nblocks // _NW)
    mesh = plsc.VectorSubcoreMesh(core_axis_name="c", subcore_axis_name="s")
    f = pl.kernel(
        functools.partial(_body, n=n, blk=blk, nblocks=nblocks, kmax=kmax),
        out_type=jax.ShapeDtypeStruct((4, n), jnp.float32),
        mesh=mesh,
        scratch_types=[pltpu.VMEM((blk,), jnp.int32) for _ in range(2)]
        + [pltpu.VMEM((4, blk), jnp.float32) for _ in range(2)]
        + [pltpu.SemaphoreType.DMA for _ in range(4)],
    )
    seq = seq.astype(jnp.int32)
    out = f(seq)
    if m < n:
        rem = seq[m:]
        cls = jnp.arange(4, dtype=jnp.int32)[:, None]
        patch = jnp.where(rem[None, :] == cls, jnp.float32(1.0),
                          jnp.where(rem[None, :] == 4,
                                    jnp.float32(0.25), jnp.float32(0.0)))
        out = lax.dynamic_update_slice(out, patch, (0, m))
    return out
